# baseline (speedup 1.0000x reference)
import functools

import jax
import jax.numpy as jnp
from jax import lax
from jax.experimental import pallas as pl
from jax.experimental.pallas import tpu as pltpu
from jax.experimental.pallas import tpu_sc as plsc

P_T = 1
T = 21
PPH = 45
PPW = 80
T_DIM = 22
H_DIM = 21
W_DIM = 21
FDH = T_DIM + H_DIM + W_DIM
GLANES = 128
PAIRS = T * PPH
ROWS = PAIRS * PPW

NC, NS = 2, 16
NW = NC * NS
K_MAX = -(-PAIRS // NW)
LAST_FULL_K = (PAIRS - (NW - 1) - 1) // NW


def _sc_body(fi_hbm, atr, ati, ahr, ahi, awr, awi, outre, outim,
             fi_v, tidx, hidx, tre, tim, hre, him, twr, twi,
             bufs_re, bufs_im, sems_re, sems_im):
    cid = lax.axis_index("c")
    sid = lax.axis_index("s")
    wid = sid * NC + cid

    pltpu.sync_copy(fi_hbm, fi_v)
    pltpu.sync_copy(awr, twr)
    pltpu.sync_copy(awi, twi)

    for g in range(2):
        k_v = lax.iota(jnp.int32, 16) + (16 * g)
        p_v = wid + NW * k_v
        p_c = jnp.where(p_v < PAIRS, p_v, 0)
        t_v = lax.div(p_c, jnp.int32(PPH))
        h_v = p_c - t_v * PPH
        f_v = plsc.load_gather(fi_v, [t_v])
        tidx[pl.ds(16 * g, 16)] = lax.div(f_v, jnp.int32(P_T))
        hidx[pl.ds(16 * g, 16)] = h_v

    g1 = pltpu.async_copy(atr.at[tidx], tre, sems_re[0])
    g2 = pltpu.async_copy(ati.at[tidx], tim, sems_im[0])
    g3 = pltpu.async_copy(ahr.at[hidx], hre, sems_re[1])
    g4 = pltpu.async_copy(ahi.at[hidx], him, sems_im[1])
    g1.wait(); g2.wait(); g3.wait(); g4.wait()

    def add_rows(r, carry):
        for g in range(4):
            sl = pl.ds(16 * g, 16)
            tre[r, sl] = tre[r, sl] + hre[r, sl]
            tim[r, sl] = tim[r, sl] + him[r, sl]
        return carry

    lax.fori_loop(0, K_MAX, add_rows, 0)

    def fill(k, b):
        pr = [tre[k, pl.ds(16 * g, 16)] for g in range(4)]
        pi = [tim[k, pl.ds(16 * g, 16)] for g in range(4)]

        def row(r, carry):
            for g in range(4):
                sl = pl.ds(16 * g, 16)
                bufs_re[b][r, sl] = twr[r, sl] + pr[g]
                bufs_im[b][r, sl] = twi[r, sl] + pi[g]
            return carry

        lax.fori_loop(0, PPW, row, 0)

    pending = [None, None]
    for k in range(K_MAX - 1):
        b = k % 2
        p = wid + NW * k
        if pending[b] is not None:
            pending[b][0].wait()
            pending[b][1].wait()
        fill(k, b)
        cre = pltpu.async_copy(bufs_re[b], outre.at[pl.ds(p * PPW, PPW)],
                               sems_re[b])
        cim = pltpu.async_copy(bufs_im[b], outim.at[pl.ds(p * PPW, PPW)],
                               sems_im[b])
        pending[b] = (cre, cim)
    for b in (0, 1):
        if pending[b] is not None:
            pending[b][0].wait()
            pending[b][1].wait()

    k_last = K_MAX - 1
    p_last = wid + NW * k_last

    def do_last():
        fill(k_last, 0)
        pltpu.sync_copy(bufs_re[0], outre.at[pl.ds(p_last * PPW, PPW)])
        pltpu.sync_copy(bufs_im[0], outim.at[pl.ds(p_last * PPW, PPW)])

    pl.when(p_last < PAIRS)(do_last)


_sc_broadcast = functools.partial(
    pl.kernel,
    out_type=[jax.ShapeDtypeStruct((ROWS, FDH), jnp.float32)] * 2,
    mesh=plsc.VectorSubcoreMesh(core_axis_name="c", subcore_axis_name="s"),
    compiler_params=pltpu.CompilerParams(needs_layout_passes=False),
    scratch_types=[
        pltpu.VMEM((32,), jnp.int32),
        pltpu.VMEM((32,), jnp.int32),
        pltpu.VMEM((32,), jnp.int32),
        pltpu.VMEM((32, GLANES), jnp.float32),
        pltpu.VMEM((32, GLANES), jnp.float32),
        pltpu.VMEM((32, GLANES), jnp.float32),
        pltpu.VMEM((32, GLANES), jnp.float32),
        pltpu.VMEM((PPW, FDH), jnp.float32),
        pltpu.VMEM((PPW, FDH), jnp.float32),
        [pltpu.VMEM((PPW, FDH), jnp.float32)] * 2,
        [pltpu.VMEM((PPW, FDH), jnp.float32)] * 2,
        [pltpu.SemaphoreType.DMA] * 2,
        [pltpu.SemaphoreType.DMA] * 2,
    ],
)(_sc_body)


def kernel(frame_indices, height, width, angles_t, angles_h, angles_w):
    del height, width
    f32 = jnp.float32
    fi = jnp.zeros((32,), jnp.int32).at[:T].set(frame_indices.astype(jnp.int32))

    nt = angles_t.shape[0]
    zt = jnp.zeros((nt, GLANES - T_DIM), f32)
    atr = jnp.concatenate([jnp.cos(angles_t), zt], axis=1)
    ati = jnp.concatenate([jnp.sin(angles_t), zt], axis=1)

    ah = angles_h[:PPH]
    zh1 = jnp.zeros((PPH, T_DIM), f32)
    zh2 = jnp.zeros((PPH, GLANES - T_DIM - H_DIM), f32)
    ahr = jnp.concatenate([zh1, jnp.cos(ah), zh2], axis=1)
    ahi = jnp.concatenate([zh1, jnp.sin(ah), zh2], axis=1)

    aw = angles_w[:PPW]
    zw = jnp.zeros((PPW, T_DIM + H_DIM), f32)
    awr = jnp.concatenate([zw, jnp.cos(aw)], axis=1)
    awi = jnp.concatenate([zw, jnp.sin(aw)], axis=1)

    re, im = _sc_broadcast(fi, atr, ati, ahr, ahi, awr, awi)
    return lax.complex(re, im).reshape(T, PPH, PPW, FDH)
